# baseline (speedup 1.0000x reference)
import functools
import math

import jax
import jax.numpy as jnp
from jax import lax
from jax.experimental import pallas as pl
from jax.experimental.pallas import tpu as pltpu
from jax.experimental.pallas import tpu_sc as plsc

VOCAB = 1000000
HIDDEN = 64
SCALE = math.sqrt(HIDDEN)

ROWS = 4096
COLS = 200
B = ROWS * COLS

NUM_CORES = 2
NUM_SUBCORES = 16
NW = NUM_CORES * NUM_SUBCORES
BPW = B // NW

CHUNK = 512
NCHUNK = BPW // CHUNK

_mesh = plsc.VectorSubcoreMesh(core_axis_name="c", subcore_axis_name="s")


@functools.partial(
    pl.kernel,
    mesh=_mesh,
    out_type=jax.ShapeDtypeStruct((B, HIDDEN), jnp.float32),
    scratch_types=[
        pltpu.VMEM((CHUNK,), jnp.int32),
        pltpu.VMEM((CHUNK, HIDDEN), jnp.float32),
        pltpu.SemaphoreType.DMA,
    ],
    compiler_params=pltpu.CompilerParams(use_tc_tiling_on_sc=False),
)
def _emb_lookup(table_hbm, idx_hbm, out_hbm, idx_v, rows_v, sem):
    wid = lax.axis_index("s") * NUM_CORES + lax.axis_index("c")
    base = wid * BPW

    def chunk_body(i, carry):
        off = base + i * CHUNK
        pltpu.sync_copy(idx_hbm.at[pl.ds(off, CHUNK)], idx_v)
        pltpu.async_copy(table_hbm.at[idx_v], rows_v, sem).wait()

        def row_body(r, c):
            for j in range(HIDDEN // 16):
                sl = pl.ds(j * 16, 16)
                rows_v[r, sl] = rows_v[r, sl] * SCALE
            return c

        lax.fori_loop(0, CHUNK, row_body, 0)
        pltpu.sync_copy(rows_v, out_hbm.at[pl.ds(off, CHUNK)])
        return carry

    lax.fori_loop(0, NCHUNK, chunk_body, 0)


def kernel(x, emb_weight):
    idx = x.reshape(B).astype(jnp.int32)
    out = _emb_lookup(emb_weight, idx)
    return out.reshape(ROWS, COLS, HIDDEN)


# speedup vs baseline: 1.1373x; 1.1373x over previous
import functools
import math

import jax
import jax.numpy as jnp
from jax import lax
from jax.experimental import pallas as pl
from jax.experimental.pallas import tpu as pltpu
from jax.experimental.pallas import tpu_sc as plsc

VOCAB = 1000000
HIDDEN = 64
SCALE = math.sqrt(HIDDEN)

ROWS = 4096
COLS = 200
B = ROWS * COLS

NUM_CORES = 2
NUM_SUBCORES = 16
NW = NUM_CORES * NUM_SUBCORES
BPW = B // NW

CHUNK = 256
NCHUNK = BPW // CHUNK
NGROUP = NCHUNK // 2

_mesh = plsc.VectorSubcoreMesh(core_axis_name="c", subcore_axis_name="s")


@functools.partial(
    pl.kernel,
    mesh=_mesh,
    out_type=jax.ShapeDtypeStruct((B, HIDDEN), jnp.float32),
    scratch_types=[
        pltpu.VMEM((BPW,), jnp.int32),
        pltpu.VMEM((CHUNK, HIDDEN), jnp.float32),
        pltpu.VMEM((CHUNK, HIDDEN), jnp.float32),
        pltpu.VMEM((CHUNK, HIDDEN), jnp.float32),
        pltpu.VMEM((CHUNK, HIDDEN), jnp.float32),
        pltpu.SemaphoreType.DMA,
        pltpu.SemaphoreType.DMA,
        pltpu.SemaphoreType.DMA,
        pltpu.SemaphoreType.DMA,
    ],
    compiler_params=pltpu.CompilerParams(use_tc_tiling_on_sc=False),
)
def _emb_lookup(table_hbm, idx_hbm, out_hbm, idx_v, g0, g1, w0, w1,
                gsem0, gsem1, wsem0, wsem1):
    wid = lax.axis_index("s") * NUM_CORES + lax.axis_index("c")
    base = wid * BPW

    def idx_slice(i):
        return idx_v.at[pl.ds(i * CHUNK, CHUNK)]

    def out_slice(i):
        return out_hbm.at[pl.ds(base + i * CHUNK, CHUNK)]

    def scale(src, dst):
        @plsc.parallel_loop(0, CHUNK, unroll=4)
        def _(r):
            for j in range(HIDDEN // 16):
                sl = pl.ds(j * 16, 16)
                dst[r, sl] = src[r, sl] * SCALE

    def step(i, gb, wb, gsem, wsem, wait_wb, issue_next):
        pltpu.make_async_copy(table_hbm.at[idx_slice(i)], gb, gsem).wait()
        if wait_wb:
            pltpu.make_async_copy(wb, out_slice(i - 2), wsem).wait()
        scale(gb, wb)
        if issue_next:
            pltpu.async_copy(table_hbm.at[idx_slice(i + 2)], gb, gsem)
        pltpu.async_copy(wb, out_slice(i), wsem)

    pltpu.sync_copy(idx_hbm.at[pl.ds(base, BPW)], idx_v)

    pltpu.async_copy(table_hbm.at[idx_slice(0)], g0, gsem0)
    pltpu.async_copy(table_hbm.at[idx_slice(1)], g1, gsem1)

    step(0, g0, w0, gsem0, wsem0, wait_wb=False, issue_next=True)
    step(1, g1, w1, gsem1, wsem1, wait_wb=False, issue_next=True)

    def group_body(g, carry):
        i = g * 2
        step(i, g0, w0, gsem0, wsem0, wait_wb=True, issue_next=True)
        step(i + 1, g1, w1, gsem1, wsem1, wait_wb=True, issue_next=True)
        return carry

    lax.fori_loop(1, NGROUP - 1, group_body, 0)

    step(NCHUNK - 2, g0, w0, gsem0, wsem0, wait_wb=True, issue_next=False)
    step(NCHUNK - 1, g1, w1, gsem1, wsem1, wait_wb=True, issue_next=False)

    pltpu.make_async_copy(w0, out_slice(NCHUNK - 2), wsem0).wait()
    pltpu.make_async_copy(w1, out_slice(NCHUNK - 1), wsem1).wait()


def kernel(x, emb_weight):
    idx = x.reshape(B).astype(jnp.int32)
    out = _emb_lookup(emb_weight, idx)
    return out.reshape(ROWS, COLS, HIDDEN)
